# Initial kernel scaffold; baseline (speedup 1.0000x reference)
import jax
import jax.numpy as jnp
from jax.experimental import pallas as pl


def kernel(local_x, global_x, local_edge_index, global_edge_index, local_edge_weight, global_edge_weight, local_params, global_params):
    raise NotImplementedError("write your pallas kernel here")


# speedup vs baseline: 9.6214x; 9.6214x over previous
import functools

import jax
import jax.numpy as jnp
from jax import lax
from jax.experimental import pallas as pl
from jax.experimental.pallas import tpu as pltpu
from jax.experimental.pallas import tpu_sc as plsc

N = 10000
D = 128
NC = 2
NS = 16
L = 16
NW = NC * NS
F = D // NW
NP = 10240
NPC = NP // NS
CHUNK = 2000


def _rsqrt16(v):
    vi = plsc.bitcast(v, jnp.int32)
    yi = jnp.int32(0x5F3759DF) - lax.shift_right_logical(vi, 1)
    y = plsc.bitcast(yi, jnp.float32)
    for _ in range(3):
        y = y * (1.5 - 0.5 * v * y * y)
    return y


def _zero_f32(ref, n):
    z = jnp.zeros((L,), jnp.float32)

    def body(i, _):
        ref[pl.ds(i * L, L)] = z
        return 0

    lax.fori_loop(0, n // L, body, 0)


def _sc_body(xtl, srcl, dstl, ewl, xtg, srcg, dstg, ewg,
             accl, dvl, accg, dvg,
             xt_v, acc_v, dinv_v, srcb, dstb, ewb, tmp, sh_part, sh_red):
    cid = lax.axis_index("c")
    sid = lax.axis_index("s")
    wid = sid * NC + cid

    def run_graph(xtH, srcH, dstH, ewH, accH, dvH):
        E = srcH.shape[0]

        _zero_f32(acc_v, F * N)
        _zero_f32(dinv_v, NP)
        epert = E // NS
        base = pl.multiple_of(sid * epert, CHUNK)

        def deg_chunk(k, _):
            off = pl.multiple_of(base + k * CHUNK, CHUNK)
            pltpu.sync_copy(dstH.at[pl.ds(off, CHUNK)], dstb)
            pltpu.sync_copy(ewH.at[pl.ds(off, CHUNK)], ewb)

            def grp(i, _):
                d16 = dstb[pl.ds(i * L, L)]
                w16 = ewb[pl.ds(i * L, L)]
                plsc.addupdate_scatter(dinv_v, [d16], w16)
                return 0

            lax.fori_loop(0, CHUNK // L, grp, 0)
            return 0

        lax.fori_loop(0, epert // CHUNK, deg_chunk, 0)

        pltpu.sync_copy(dinv_v, sh_part.at[sid])
        plsc.subcore_barrier()

        cbase = pl.multiple_of(sid * NPC, NPC)
        pltpu.sync_copy(sh_part.at[0, pl.ds(cbase, NPC)], tmp.at[pl.ds(0, NPC)])
        for t in range(1, NS):
            pltpu.sync_copy(sh_part.at[t, pl.ds(cbase, NPC)],
                            tmp.at[pl.ds(NPC, NPC)])

            def addb(i, _):
                tmp[pl.ds(i * L, L)] += tmp[pl.ds(NPC + i * L, L)]
                return 0

            lax.fori_loop(0, NPC // L, addb, 0)

        def nwt(i, _):
            v = tmp[pl.ds(i * L, L)] + 1.0
            tmp[pl.ds(i * L, L)] = _rsqrt16(v)
            return 0

        lax.fori_loop(0, NPC // L, nwt, 0)
        pltpu.sync_copy(tmp.at[pl.ds(0, NPC)], sh_red.at[pl.ds(cbase, NPC)])
        plsc.subcore_barrier()
        pltpu.sync_copy(sh_red, dinv_v)

        @pl.when(jnp.logical_and(sid == 0, cid == 0))
        def _():
            pltpu.sync_copy(dinv_v.at[pl.ds(0, N)], dvH)

        xoff = pl.multiple_of(wid * (F * N), F * N)
        pltpu.sync_copy(xtH.at[pl.ds(xoff, F * N)], xt_v)

        def edge_chunk(k, _):
            off = pl.multiple_of(k * CHUNK, CHUNK)
            pltpu.sync_copy(srcH.at[pl.ds(off, CHUNK)], srcb)
            pltpu.sync_copy(dstH.at[pl.ds(off, CHUNK)], dstb)
            pltpu.sync_copy(ewH.at[pl.ds(off, CHUNK)], ewb)

            def grp(i, _):
                s16 = srcb[pl.ds(i * L, L)]
                d16 = dstb[pl.ds(i * L, L)]
                w16 = ewb[pl.ds(i * L, L)]
                coef = w16 * plsc.load_gather(dinv_v, [s16])
                for d in range(F):
                    xv = plsc.load_gather(xt_v, [s16 + d * N])
                    plsc.addupdate_scatter(acc_v, [d16 + d * N], xv * coef)
                return 0

            lax.fori_loop(0, CHUNK // L, grp, 0)
            return 0

        lax.fori_loop(0, E // CHUNK, edge_chunk, 0)
        pltpu.sync_copy(acc_v, accH.at[pl.ds(xoff, F * N)])

    run_graph(xtl, srcl, dstl, ewl, accl, dvl)
    run_graph(xtg, srcg, dstg, ewg, accg, dvg)


_sc_propagate = pl.kernel(
    _sc_body,
    out_type=[
        jax.ShapeDtypeStruct((D * N,), jnp.float32),
        jax.ShapeDtypeStruct((N,), jnp.float32),
        jax.ShapeDtypeStruct((D * N,), jnp.float32),
        jax.ShapeDtypeStruct((N,), jnp.float32),
    ],
    mesh=plsc.VectorSubcoreMesh(core_axis_name="c", subcore_axis_name="s"),
    compiler_params=pltpu.CompilerParams(needs_layout_passes=False),
    scratch_types=[
        pltpu.VMEM((F * N,), jnp.float32),
        pltpu.VMEM((F * N,), jnp.float32),
        pltpu.VMEM((NP,), jnp.float32),
        pltpu.VMEM((CHUNK,), jnp.int32),
        pltpu.VMEM((CHUNK,), jnp.int32),
        pltpu.VMEM((CHUNK,), jnp.float32),
        pltpu.VMEM((2 * NPC,), jnp.float32),
        pltpu.VMEM_SHARED((NS, NP), jnp.float32),
        pltpu.VMEM_SHARED((NP,), jnp.float32),
    ],
)


def _tc_body(xT, accT, dinv, Wz, bz, Lz, lzb, Wh, bh, Lh, lhb, out):
    dv = dinv[...]
    yT = accT[...] * dv + (dv * dv) * xT[...]
    Lzt = Lz[...][:D, :]
    Mz = jnp.dot(Wz[...], Lzt, preferred_element_type=jnp.float32)
    czb = jnp.dot(bz[...], Lzt, preferred_element_type=jnp.float32) + lzb[...]
    Lht = Lh[...][:D, :]
    Mh = jnp.dot(Wh[...], Lht, preferred_element_type=jnp.float32)
    chb = jnp.dot(bh[...], Lht, preferred_element_type=jnp.float32) + lhb[...]
    z = lax.dot_general(yT, Mz, (((0,), (0,)), ((), ())),
                        preferred_element_type=jnp.float32)
    h = lax.dot_general(yT, Mh, (((0,), (0,)), ((), ())),
                        preferred_element_type=jnp.float32)
    out[...] = (1.0 - jax.nn.sigmoid(z + czb)) * jnp.tanh(h + chb)


_tc_combine = pl.pallas_call(
    _tc_body,
    out_shape=jax.ShapeDtypeStruct((N, D), jnp.float32),
)


def _dense_tail(xt, acc, dv, params):
    Wz, bz, Lz, lzb, _Wr, _br, _Lr, _lrb, Wh, bh, Lh, lhb = params
    return _tc_combine(
        xt.reshape(D, N), acc.reshape(D, N), dv.reshape(1, N),
        Wz, bz.reshape(1, D), Lz, lzb.reshape(1, D),
        Wh, bh.reshape(1, D), Lh, lhb.reshape(1, D))


def kernel(local_x, global_x, local_edge_index, global_edge_index,
           local_edge_weight, global_edge_weight, local_params, global_params):
    xtl = local_x.T.reshape(-1)
    xtg = global_x.T.reshape(-1)
    accl, dvl, accg, dvg = _sc_propagate(
        xtl, local_edge_index[0], local_edge_index[1], local_edge_weight,
        xtg, global_edge_index[0], global_edge_index[1], global_edge_weight)
    out_l = _dense_tail(xtl, accl, dvl, local_params)
    out_g = _dense_tail(xtg, accg, dvg, global_params)
    return (out_l, out_g)


# speedup vs baseline: 32.6336x; 3.3918x over previous
import functools

import jax
import jax.numpy as jnp
from jax import lax
from jax.experimental import pallas as pl
from jax.experimental.pallas import tpu as pltpu
from jax.experimental.pallas import tpu_sc as plsc

N = 10000
D = 128
NC = 2
NS = 16
L = 16
NW = NC * NS
F = D // NW
NP = 10240
NPC = NP // NS
CHUNK = 2000


def _rsqrt16(v):
    vi = plsc.bitcast(v, jnp.int32)
    yi = jnp.int32(0x5F3759DF) - lax.shift_right_logical(vi, 1)
    y = plsc.bitcast(yi, jnp.float32)
    for _ in range(3):
        y = y * (1.5 - 0.5 * v * y * y)
    return y


def _zero_f32(ref, n):
    z = jnp.zeros((L,), jnp.float32)

    @plsc.parallel_loop(0, n // L, 1, unroll=4)
    def _(i):
        ref[pl.ds(i * L, L)] = z


def _sc_body(xtl, srcl, dstl, ewl, xtg, srcg, dstg, ewg,
             accl, dvl, accg, dvg,
             xt_v, acc_v, dinv_v, srcb, dstb, ewb, tmp, sh_part, sh_red, sems):
    cid = lax.axis_index("c")
    sid = lax.axis_index("s")
    wid = sid * NC + cid

    def run_graph(xtH, srcH, dstH, ewH, accH, dvH):
        E = srcH.shape[0]

        _zero_f32(acc_v, F * N)
        _zero_f32(dinv_v, NP)
        epert = E // NS
        base = pl.multiple_of(sid * epert, CHUNK)

        def deg_chunk(k, _):
            off = pl.multiple_of(base + k * CHUNK, CHUNK)
            pltpu.sync_copy(dstH.at[pl.ds(off, CHUNK)], dstb.at[pl.ds(0, CHUNK)])
            pltpu.sync_copy(ewH.at[pl.ds(off, CHUNK)], ewb.at[pl.ds(0, CHUNK)])

            @plsc.parallel_loop(0, CHUNK // L, 1, unroll=4)
            def _(i):
                d16 = dstb[pl.ds(i * L, L)]
                w16 = ewb[pl.ds(i * L, L)]
                plsc.addupdate_scatter(dinv_v, [d16], w16)

            return 0

        lax.fori_loop(0, epert // CHUNK, deg_chunk, 0)

        pltpu.sync_copy(dinv_v, sh_part.at[sid])
        plsc.subcore_barrier()

        cbase = pl.multiple_of(sid * NPC, NPC)
        pltpu.sync_copy(sh_part.at[0, pl.ds(cbase, NPC)], tmp.at[pl.ds(0, NPC)])
        for t in range(1, NS):
            pltpu.sync_copy(sh_part.at[t, pl.ds(cbase, NPC)],
                            tmp.at[pl.ds(NPC, NPC)])

            @plsc.parallel_loop(0, NPC // L, 1, unroll=4)
            def _(i):
                tmp[pl.ds(i * L, L)] += tmp[pl.ds(NPC + i * L, L)]

        @plsc.parallel_loop(0, NPC // L, 1, unroll=2)
        def _(i):
            v = tmp[pl.ds(i * L, L)] + 1.0
            tmp[pl.ds(i * L, L)] = _rsqrt16(v)

        pltpu.sync_copy(tmp.at[pl.ds(0, NPC)], sh_red.at[pl.ds(cbase, NPC)])
        plsc.subcore_barrier()
        pltpu.sync_copy(sh_red, dinv_v)

        @pl.when(jnp.logical_and(sid == 0, cid == 0))
        def _():
            pltpu.sync_copy(dinv_v.at[pl.ds(0, N)], dvH)

        xoff = pl.multiple_of(wid * (F * N), F * N)
        pltpu.sync_copy(xtH.at[pl.ds(xoff, F * N)], xt_v)
        for d in range(F):

            @plsc.parallel_loop(0, N // L, 1, unroll=4)
            def _(i):
                xt_v[pl.ds(d * N + i * L, L)] *= dinv_v[pl.ds(i * L, L)]

        nch = E // CHUNK

        def issue(k, slot):
            off = pl.multiple_of(k * CHUNK, CHUNK)
            sl = pl.ds(slot * CHUNK, CHUNK)
            pltpu.async_copy(srcH.at[pl.ds(off, CHUNK)], srcb.at[sl],
                             sems.at[slot])
            pltpu.async_copy(dstH.at[pl.ds(off, CHUNK)], dstb.at[sl],
                             sems.at[slot])
            pltpu.async_copy(ewH.at[pl.ds(off, CHUNK)], ewb.at[sl],
                             sems.at[slot])

        def drain(slot):
            sl = pl.ds(slot * CHUNK, CHUNK)
            pltpu.make_async_copy(srcH.at[pl.ds(0, CHUNK)], srcb.at[sl],
                                  sems.at[slot]).wait()
            pltpu.make_async_copy(dstH.at[pl.ds(0, CHUNK)], dstb.at[sl],
                                  sems.at[slot]).wait()
            pltpu.make_async_copy(ewH.at[pl.ds(0, CHUNK)], ewb.at[sl],
                                  sems.at[slot]).wait()

        def process(slot):
            @plsc.parallel_loop(0, CHUNK // L, 1, unroll=2)
            def _(i):
                s16 = srcb[pl.ds(slot * CHUNK + i * L, L)]
                d16 = dstb[pl.ds(slot * CHUNK + i * L, L)]
                w16 = ewb[pl.ds(slot * CHUNK + i * L, L)]
                for d in range(F):
                    xv = plsc.load_gather(xt_v, [s16 + d * N])
                    plsc.addupdate_scatter(acc_v, [d16 + d * N], xv * w16)

        issue(0, 0)

        def pair(kk, _):
            for slot in (0, 1):
                k = kk * 2 + slot

                @pl.when(k + 1 < nch)
                def _():
                    issue(k + 1, 1 - slot)

                drain(slot)
                process(slot)
            return 0

        lax.fori_loop(0, nch // 2, pair, 0)
        pltpu.sync_copy(acc_v, accH.at[pl.ds(xoff, F * N)])

    run_graph(xtl, srcl, dstl, ewl, accl, dvl)
    run_graph(xtg, srcg, dstg, ewg, accg, dvg)


_sc_propagate = pl.kernel(
    _sc_body,
    out_type=[
        jax.ShapeDtypeStruct((D * N,), jnp.float32),
        jax.ShapeDtypeStruct((N,), jnp.float32),
        jax.ShapeDtypeStruct((D * N,), jnp.float32),
        jax.ShapeDtypeStruct((N,), jnp.float32),
    ],
    mesh=plsc.VectorSubcoreMesh(core_axis_name="c", subcore_axis_name="s"),
    compiler_params=pltpu.CompilerParams(needs_layout_passes=False),
    scratch_types=[
        pltpu.VMEM((F * N,), jnp.float32),
        pltpu.VMEM((F * N,), jnp.float32),
        pltpu.VMEM((NP,), jnp.float32),
        pltpu.VMEM((2 * CHUNK,), jnp.int32),
        pltpu.VMEM((2 * CHUNK,), jnp.int32),
        pltpu.VMEM((2 * CHUNK,), jnp.float32),
        pltpu.VMEM((2 * NPC,), jnp.float32),
        pltpu.VMEM_SHARED((NS, NP), jnp.float32),
        pltpu.VMEM_SHARED((NP,), jnp.float32),
        pltpu.SemaphoreType.DMA((2,)),
    ],
)


def _tc_body(xT, accT, dinv, Wz, bz, Lz, lzb, Wh, bh, Lh, lhb, out):
    dv = dinv[...]
    yT = accT[...] * dv + (dv * dv) * xT[...]
    Lzt = Lz[...][:D, :]
    Mz = jnp.dot(Wz[...], Lzt, preferred_element_type=jnp.float32)
    czb = jnp.dot(bz[...], Lzt, preferred_element_type=jnp.float32) + lzb[...]
    Lht = Lh[...][:D, :]
    Mh = jnp.dot(Wh[...], Lht, preferred_element_type=jnp.float32)
    chb = jnp.dot(bh[...], Lht, preferred_element_type=jnp.float32) + lhb[...]
    z = lax.dot_general(yT, Mz, (((0,), (0,)), ((), ())),
                        preferred_element_type=jnp.float32)
    h = lax.dot_general(yT, Mh, (((0,), (0,)), ((), ())),
                        preferred_element_type=jnp.float32)
    out[...] = (1.0 - jax.nn.sigmoid(z + czb)) * jnp.tanh(h + chb)


_tc_combine = pl.pallas_call(
    _tc_body,
    out_shape=jax.ShapeDtypeStruct((N, D), jnp.float32),
)


def _dense_tail(xt, acc, dv, params):
    Wz, bz, Lz, lzb, _Wr, _br, _Lr, _lrb, Wh, bh, Lh, lhb = params
    return _tc_combine(
        xt.reshape(D, N), acc.reshape(D, N), dv.reshape(1, N),
        Wz, bz.reshape(1, D), Lz, lzb.reshape(1, D),
        Wh, bh.reshape(1, D), Lh, lhb.reshape(1, D))


def kernel(local_x, global_x, local_edge_index, global_edge_index,
           local_edge_weight, global_edge_weight, local_params, global_params):
    xtl = local_x.T.reshape(-1)
    xtg = global_x.T.reshape(-1)
    accl, dvl, accg, dvg = _sc_propagate(
        xtl, local_edge_index[0], local_edge_index[1], local_edge_weight,
        xtg, global_edge_index[0], global_edge_index[1], global_edge_weight)
    out_l = _dense_tail(xtl, accl, dvl, local_params)
    out_g = _dense_tail(xtg, accg, dvg, global_params)
    return (out_l, out_g)


# speedup vs baseline: 33.9009x; 1.0388x over previous
import functools

import jax
import jax.numpy as jnp
from jax import lax
from jax.experimental import pallas as pl
from jax.experimental.pallas import tpu as pltpu
from jax.experimental.pallas import tpu_sc as plsc

N = 10000
D = 128
NC = 2
NS = 16
L = 16
NW = NC * NS
F = D // NW
NP = 10240
NPC = NP // NS
CHUNK = 3200
DCH = 2000


def _rsqrt16(v):
    vi = plsc.bitcast(v, jnp.int32)
    yi = jnp.int32(0x5F3759DF) - lax.shift_right_logical(vi, 1)
    y = plsc.bitcast(yi, jnp.float32)
    for _ in range(3):
        y = y * (1.5 - 0.5 * v * y * y)
    return y


def _zero_f32(ref, n):
    z = jnp.zeros((L,), jnp.float32)

    @plsc.parallel_loop(0, n // L, 1, unroll=4)
    def _(i):
        ref[pl.ds(i * L, L)] = z


def _sc_body(xtl, srcl, dstl, ewl, xtg, srcg, dstg, ewg,
             accl, accg,
             xt_v, acc_v, dinv_v, srcb, dstb, ewb, tmp, sh_part, sh_red, sems):
    cid = lax.axis_index("c")
    sid = lax.axis_index("s")
    wid = sid * NC + cid

    def run_graph(xtH, srcH, dstH, ewH, accH):
        E = srcH.shape[0]

        _zero_f32(dinv_v, NP)
        epert = E // NS
        base = pl.multiple_of(sid * epert, DCH)

        def deg_chunk(k, _):
            off = pl.multiple_of(base + k * DCH, DCH)
            pltpu.sync_copy(dstH.at[pl.ds(off, DCH)], dstb.at[pl.ds(0, DCH)])
            pltpu.sync_copy(ewH.at[pl.ds(off, DCH)], ewb.at[pl.ds(0, DCH)])

            @plsc.parallel_loop(0, DCH // L, 1, unroll=4)
            def _(i):
                d16 = dstb[pl.ds(i * L, L)]
                w16 = ewb[pl.ds(i * L, L)]
                plsc.addupdate_scatter(dinv_v, [d16], w16)

            return 0

        lax.fori_loop(0, epert // DCH, deg_chunk, 0)

        pltpu.sync_copy(dinv_v, sh_part.at[sid])
        plsc.subcore_barrier()

        cbase = pl.multiple_of(sid * NPC, NPC)
        pltpu.sync_copy(sh_part.at[0, pl.ds(cbase, NPC)], tmp.at[pl.ds(0, NPC)])
        for t in range(1, NS):
            pltpu.sync_copy(sh_part.at[t, pl.ds(cbase, NPC)],
                            tmp.at[pl.ds(NPC, NPC)])

            @plsc.parallel_loop(0, NPC // L, 1, unroll=4)
            def _(i):
                tmp[pl.ds(i * L, L)] += tmp[pl.ds(NPC + i * L, L)]

        @plsc.parallel_loop(0, NPC // L, 1, unroll=2)
        def _(i):
            v = tmp[pl.ds(i * L, L)] + 1.0
            tmp[pl.ds(i * L, L)] = _rsqrt16(v)

        pltpu.sync_copy(tmp.at[pl.ds(0, NPC)], sh_red.at[pl.ds(cbase, NPC)])
        plsc.subcore_barrier()
        pltpu.sync_copy(sh_red, dinv_v)

        xoff = pl.multiple_of(wid * (F * N), F * N)
        pltpu.sync_copy(xtH.at[pl.ds(xoff, F * N)], xt_v)
        for d in range(F):

            @plsc.parallel_loop(0, N // L, 1, unroll=4)
            def _(i):
                u = xt_v[pl.ds(d * N + i * L, L)] * dinv_v[pl.ds(i * L, L)]
                xt_v[pl.ds(d * N + i * L, L)] = u
                acc_v[pl.ds(d * N + i * L, L)] = u

        nch = E // CHUNK

        def issue(k, slot):
            off = pl.multiple_of(k * CHUNK, CHUNK)
            sl = pl.ds(slot * CHUNK, CHUNK)
            pltpu.async_copy(srcH.at[pl.ds(off, CHUNK)], srcb.at[sl],
                             sems.at[slot])
            pltpu.async_copy(dstH.at[pl.ds(off, CHUNK)], dstb.at[sl],
                             sems.at[slot])
            pltpu.async_copy(ewH.at[pl.ds(off, CHUNK)], ewb.at[sl],
                             sems.at[slot])

        def drain(slot):
            sl = pl.ds(slot * CHUNK, CHUNK)
            pltpu.make_async_copy(srcH.at[pl.ds(0, CHUNK)], srcb.at[sl],
                                  sems.at[slot]).wait()
            pltpu.make_async_copy(dstH.at[pl.ds(0, CHUNK)], dstb.at[sl],
                                  sems.at[slot]).wait()
            pltpu.make_async_copy(ewH.at[pl.ds(0, CHUNK)], ewb.at[sl],
                                  sems.at[slot]).wait()

        def process(slot):
            @plsc.parallel_loop(0, CHUNK // L, 1, unroll=4)
            def _(i):
                s16 = srcb[pl.ds(slot * CHUNK + i * L, L)]
                d16 = dstb[pl.ds(slot * CHUNK + i * L, L)]
                w16 = ewb[pl.ds(slot * CHUNK + i * L, L)]
                for d in range(F):
                    xv = plsc.load_gather(xt_v, [s16 + d * N])
                    plsc.addupdate_scatter(acc_v, [d16 + d * N], xv * w16)

        issue(0, 0)

        def pair(kk, _):
            for slot in (0, 1):
                k = kk * 2 + slot

                @pl.when(k + 1 < nch)
                def _():
                    issue(k + 1, 1 - slot)

                drain(slot)
                process(slot)
            return 0

        lax.fori_loop(0, nch // 2, pair, 0)

        for d in range(F):

            @plsc.parallel_loop(0, N // L, 1, unroll=4)
            def _(i):
                acc_v[pl.ds(d * N + i * L, L)] *= dinv_v[pl.ds(i * L, L)]

        pltpu.sync_copy(acc_v, accH.at[pl.ds(xoff, F * N)])

    run_graph(xtl, srcl, dstl, ewl, accl)
    run_graph(xtg, srcg, dstg, ewg, accg)


_sc_propagate = pl.kernel(
    _sc_body,
    out_type=[
        jax.ShapeDtypeStruct((D * N,), jnp.float32),
        jax.ShapeDtypeStruct((D * N,), jnp.float32),
    ],
    mesh=plsc.VectorSubcoreMesh(core_axis_name="c", subcore_axis_name="s"),
    compiler_params=pltpu.CompilerParams(needs_layout_passes=False),
    scratch_types=[
        pltpu.VMEM((F * N,), jnp.float32),
        pltpu.VMEM((F * N,), jnp.float32),
        pltpu.VMEM((NP,), jnp.float32),
        pltpu.VMEM((2 * CHUNK,), jnp.int32),
        pltpu.VMEM((2 * CHUNK,), jnp.int32),
        pltpu.VMEM((2 * CHUNK,), jnp.float32),
        pltpu.VMEM((2 * NPC,), jnp.float32),
        pltpu.VMEM_SHARED((NS, NP), jnp.float32),
        pltpu.VMEM_SHARED((NP,), jnp.float32),
        pltpu.SemaphoreType.DMA((2,)),
    ],
)


def _gates(yT, Wz, bz, Lz, lzb, Wh, bh, Lh, lhb):
    Lzt = Lz[...][:D, :]
    Mz = jnp.dot(Wz[...], Lzt, preferred_element_type=jnp.float32)
    czb = jnp.dot(bz[...], Lzt, preferred_element_type=jnp.float32) + lzb[...]
    Lht = Lh[...][:D, :]
    Mh = jnp.dot(Wh[...], Lht, preferred_element_type=jnp.float32)
    chb = jnp.dot(bh[...], Lht, preferred_element_type=jnp.float32) + lhb[...]
    z = lax.dot_general(yT, Mz, (((0,), (0,)), ((), ())),
                        preferred_element_type=jnp.float32)
    h = lax.dot_general(yT, Mh, (((0,), (0,)), ((), ())),
                        preferred_element_type=jnp.float32)
    return (1.0 - jax.nn.sigmoid(z + czb)) * jnp.tanh(h + chb)


def _tc_body(ytl, ytg,
             Wzl, bzl, Lzl, lzbl, Whl, bhl, Lhl, lhbl,
             Wzg, bzg, Lzg, lzbg, Whg, bhg, Lhg, lhbg,
             outl, outg):
    outl[...] = _gates(ytl[...], Wzl, bzl, Lzl, lzbl, Whl, bhl, Lhl, lhbl)
    outg[...] = _gates(ytg[...], Wzg, bzg, Lzg, lzbg, Whg, bhg, Lhg, lhbg)


_tc_combine = pl.pallas_call(
    _tc_body,
    out_shape=[jax.ShapeDtypeStruct((N, D), jnp.float32),
               jax.ShapeDtypeStruct((N, D), jnp.float32)],
)


def _param_block(params):
    Wz, bz, Lz, lzb, _Wr, _br, _Lr, _lrb, Wh, bh, Lh, lhb = params
    return (Wz, bz.reshape(1, D), Lz, lzb.reshape(1, D),
            Wh, bh.reshape(1, D), Lh, lhb.reshape(1, D))


def kernel(local_x, global_x, local_edge_index, global_edge_index,
           local_edge_weight, global_edge_weight, local_params, global_params):
    xtl = local_x.T.reshape(-1)
    xtg = global_x.T.reshape(-1)
    ytl, ytg = _sc_propagate(
        xtl, local_edge_index[0], local_edge_index[1], local_edge_weight,
        xtg, global_edge_index[0], global_edge_index[1], global_edge_weight)
    out_l, out_g = _tc_combine(
        ytl.reshape(D, N), ytg.reshape(D, N),
        *_param_block(local_params), *_param_block(global_params))
    return (out_l, out_g)
